# Initial kernel scaffold; baseline (speedup 1.0000x reference)
import jax
import jax.numpy as jnp
from jax.experimental import pallas as pl


def kernel(x, batch_embed, token_embed):
    raise NotImplementedError("write your pallas kernel here")


# speedup vs baseline: 14.0526x; 14.0526x over previous
import jax
import jax.numpy as jnp
from jax.experimental import pallas as pl


_BB = 128


def _body(b_ref, t_ref, o_ref):
    o_ref[...] = b_ref[...][:, None, :] + t_ref[...][None, :, :]


def kernel(x, batch_embed, token_embed):
    del x
    B, D = batch_embed.shape
    T = token_embed.shape[0]
    grid = (B // _BB,)
    return pl.pallas_call(
        _body,
        grid=grid,
        in_specs=[
            pl.BlockSpec((_BB, D), lambda i: (i, 0)),
            pl.BlockSpec((T, D), lambda i: (0, 0)),
        ],
        out_specs=pl.BlockSpec((_BB, T, D), lambda i: (i, 0, 0)),
        out_shape=jax.ShapeDtypeStruct((B, T, D), jnp.float32),
    )(batch_embed, token_embed)


# speedup vs baseline: 86.9410x; 6.1868x over previous
import jax
import jax.numpy as jnp
from jax.experimental import pallas as pl


_TB = 8


def _body(t_ref, bt_ref, o_ref):
    o_ref[...] = t_ref[...][:, :, None] + bt_ref[...][None, :, :]


def kernel(x, batch_embed, token_embed):
    del x
    B, D = batch_embed.shape
    T = token_embed.shape[0]
    bt = batch_embed.T
    out_p = pl.pallas_call(
        _body,
        grid=(T // _TB,),
        in_specs=[
            pl.BlockSpec((_TB, D), lambda i: (i, 0)),
            pl.BlockSpec((D, B), lambda i: (0, 0)),
        ],
        out_specs=pl.BlockSpec((_TB, D, B), lambda i: (i, 0, 0)),
        out_shape=jax.ShapeDtypeStruct((T, D, B), jnp.float32),
    )(token_embed, bt)
    return jnp.transpose(out_p, (2, 0, 1))
